# baseline (speedup 1.0000x reference)
import functools

import jax
import jax.numpy as jnp
from jax import lax
from jax.experimental import pallas as pl
from jax.experimental.pallas import tpu as pltpu
from jax.experimental.pallas import tpu_sc as plsc

N, H, W, K, F, C = 4, 512, 512, 8, 100000, 16
B = N * H * W
NW = 32
PXW = B // NW
P = 1024
NCH = PXW // P
GSZ = 128
NG = P // GSZ
L = 16


def _tex_kernel(pix_hbm, bary_hbm, table_hbm, out_hbm,
                pix8_v, bary_v, idx_v, rows_v, out_v, semb, semg):
    wid = lax.axis_index("s") * 2 + lax.axis_index("c")
    iota = lax.iota(jnp.int32, L)
    iota8 = iota * 8

    def chunk_body(ci, carry):
        base = pl.multiple_of(wid * PXW + ci * P, P)

        bary_cp = pltpu.async_copy(
            bary_hbm.at[pl.ds(base * 24, P * 24)], bary_v, semb)
        pltpu.sync_copy(pix_hbm.at[pl.ds(base * 8, P * 8)], pix8_v)

        gather_cps = []
        for j in range(NG):
            def ex_body(g, _, j=j):
                pvi = jnp.full((L,), 8 * (j * GSZ + g * L), jnp.int32) + iota8
                idx_v[j, pl.ds(g * L, L)] = plsc.load_gather(pix8_v, [pvi])
                return 0
            lax.fori_loop(0, GSZ // L, ex_body, 0)
            gather_cps.append(
                pltpu.async_copy(table_hbm.at[idx_v.at[j]], rows_v.at[j], semg))

        bary_cp.wait()
        for cp in gather_cps:
            cp.wait()

        for j in range(NG):
            def px_body(g, _, j=j):
                pq = jnp.full((L,), j * GSZ + g * L, jnp.int32) + iota
                pin = jnp.full((L,), g * L, jnp.int32) + iota
                p24 = pq * 24
                p16 = pq * 16
                b0 = plsc.load_gather(bary_v, [p24])
                b1 = plsc.load_gather(bary_v, [p24 + 1])
                b2 = plsc.load_gather(bary_v, [p24 + 2])
                for c in range(C):
                    r0 = plsc.load_gather(
                        rows_v, [jnp.full((L,), j, jnp.int32), pin,
                                 jnp.full((L,), c, jnp.int32)])
                    r1 = plsc.load_gather(
                        rows_v, [jnp.full((L,), j, jnp.int32), pin,
                                 jnp.full((L,), 16 + c, jnp.int32)])
                    r2 = plsc.load_gather(
                        rows_v, [jnp.full((L,), j, jnp.int32), pin,
                                 jnp.full((L,), 32 + c, jnp.int32)])
                    acc = b0 * r0 + b1 * r1 + b2 * r2
                    plsc.store_scatter(out_v, [p16 + c], acc)
                return 0
            lax.fori_loop(0, GSZ // L, px_body, 0)

        pltpu.sync_copy(out_v, out_hbm.at[pl.ds(base * 16, P * 16)])
        return carry

    lax.fori_loop(0, NCH, chunk_body, 0)


@jax.jit
def _run(pix, bary, table):
    mesh = plsc.VectorSubcoreMesh(core_axis_name="c", subcore_axis_name="s")
    f = functools.partial(
        pl.kernel,
        mesh=mesh,
        compiler_params=pltpu.CompilerParams(
            needs_layout_passes=False, use_tc_tiling_on_sc=False),
        out_type=jax.ShapeDtypeStruct((B * 16,), jnp.float32),
        scratch_types=[
            pltpu.VMEM((P * 8,), jnp.int32),
            pltpu.VMEM((P * 24,), jnp.float32),
            pltpu.VMEM((NG, GSZ), jnp.int32),
            pltpu.VMEM((NG, GSZ, 3 * C), jnp.float32),
            pltpu.VMEM((P * 16,), jnp.float32),
            pltpu.SemaphoreType.DMA,
            pltpu.SemaphoreType.DMA,
        ],
    )(_tex_kernel)
    return f(pix, bary, table)


def kernel(bary_coords, pix_to_face, face_textures):
    pix = pix_to_face.astype(jnp.int32).reshape(B * K)
    bary = bary_coords.reshape(B * K * 3)
    table = face_textures.reshape(F, 3 * C)
    out = _run(pix, bary, table)
    return out.reshape(N, H, W, C)
